# baseline (speedup 1.0000x reference)
import functools

import jax
import jax.numpy as jnp
from jax import lax
from jax.experimental import pallas as pl
from jax.experimental.pallas import tpu as pltpu
from jax.experimental.pallas import tpu_sc as plsc

N_NODES = 10000
N_EDGES = 320000
D_FEAT = 128
L = 16

_INFO = plsc.get_sparse_core_info()
NC = _INFO.num_cores
NS = _INFO.num_subcores
NW = NC * NS
E_PER_W = N_EDGES // NW
CHUNK = 80
N_CHUNKS = E_PER_W // CHUNK
N_BLK = D_FEAT // L

_mesh = plsc.VectorSubcoreMesh(core_axis_name="c", subcore_axis_name="s")


@functools.partial(
    pl.kernel,
    out_type=jax.ShapeDtypeStruct((N_EDGES,), jnp.float32),
    mesh=_mesh,
    scratch_types=[
        pltpu.VMEM((CHUNK,), jnp.int32),
        pltpu.VMEM((CHUNK,), jnp.int32),
        pltpu.VMEM((CHUNK, D_FEAT), jnp.float32),
        pltpu.VMEM((CHUNK, D_FEAT), jnp.float32),
        pltpu.VMEM((CHUNK,), jnp.float32),
        pltpu.VMEM((D_FEAT,), jnp.float32),
        pltpu.VMEM((L,), jnp.float32),
        pltpu.SemaphoreType.DMA,
        pltpu.SemaphoreType.DMA,
    ],
    compiler_params=pltpu.CompilerParams(needs_layout_passes=False),
)
def _decoder_sc(table, ridx_hbm, cidx_hbm, w_hbm, binit_hbm, out_hbm,
                ridx_v, cidx_v, rows_r, rows_c, out_v, w_v, binit_v,
                sem_r, sem_c):
    wid = lax.axis_index("s") * NC + lax.axis_index("c")
    base = wid * E_PER_W
    pltpu.sync_copy(w_hbm, w_v)
    pltpu.sync_copy(binit_hbm, binit_v)
    b_reg = binit_v[...]
    lane = lax.iota(jnp.int32, L)
    w_regs = [w_v[pl.ds(i * L, L)] for i in range(N_BLK)]

    def chunk_body(ci, carry):
        off = base + ci * CHUNK
        pltpu.sync_copy(ridx_hbm.at[pl.ds(off, CHUNK)], ridx_v)
        pltpu.sync_copy(cidx_hbm.at[pl.ds(off, CHUNK)], cidx_v)
        r_cp = pltpu.async_copy(table.at[ridx_v], rows_r, sem_r)
        c_cp = pltpu.async_copy(table.at[cidx_v], rows_c, sem_c)
        r_cp.wait()
        c_cp.wait()

        def group_body(g, ecarry):
            erow = g * L + lane
            acc = b_reg
            for d in range(D_FEAT):
                dcol = jnp.full((L,), d, jnp.int32)
                rv = plsc.load_gather(rows_r, [erow, dcol])
                cv = plsc.load_gather(rows_c, [erow, dcol])
                acc = acc + jnp.abs(rv - cv) * w_regs[d // L][d % L]
            out_v[pl.ds(g * L, L)] = acc
            return ecarry

        lax.fori_loop(0, CHUNK // L, group_body, 0)
        pltpu.sync_copy(out_v, out_hbm.at[pl.ds(off, CHUNK)])
        return carry

    lax.fori_loop(0, N_CHUNKS, chunk_body, 0)


def kernel(inputs, r_indices, c_indices, W, b):
    w_flat = W.reshape(D_FEAT)
    binit = jnp.full((L,), b[0], jnp.float32)
    out = _decoder_sc(inputs, r_indices, c_indices, w_flat, binit)
    return out.reshape(N_EDGES, 1)
